# baseline (speedup 1.0000x reference)
import functools

import jax
import jax.numpy as jnp
from jax.experimental import pallas as pl
from jax.experimental.pallas import tpu as pltpu
from jax.experimental.pallas import tpu_sc as plsc

_K = 256
_D = 8192
_NV = _D // 16
_INT_MIN = -(2**31)
_TOPBIT = 0x80000000

_TC_ROWS = 2048


def _tc_topk_kernel(x_ref, o_ref):
    x = x_ref[...]
    b = jax.lax.bitcast_convert_type(x, jnp.int32)
    keys = jnp.where(b < 0, b ^ jnp.int32(0x7FFFFFFF), b)

    rows = x.shape[0]

    def body(j, u):
        bit = jnp.left_shift(jnp.int32(1), jnp.int32(31) - j)
        cand_u = u | bit
        cand_s = cand_u ^ jnp.int32(_INT_MIN)
        cnt = jnp.sum((keys >= cand_s).astype(jnp.int32), axis=1, keepdims=True)
        return jnp.where(cnt >= _K, cand_u, u)

    u0 = jnp.zeros((rows, 1), jnp.int32)
    u_star = jax.lax.fori_loop(0, 32, body, u0)
    thr = u_star ^ jnp.int32(_INT_MIN)
    o_ref[...] = jnp.where(keys >= thr, x, jnp.float32(0.0))


def _tc_topk(x):
    rows_per_block = 128
    while x.shape[0] % rows_per_block:
        rows_per_block //= 2
    grid = (x.shape[0] // rows_per_block,)
    return pl.pallas_call(
        _tc_topk_kernel,
        out_shape=jax.ShapeDtypeStruct(x.shape, x.dtype),
        grid=grid,
        in_specs=[pl.BlockSpec((rows_per_block, _D), lambda i: (i, 0))],
        out_specs=pl.BlockSpec((rows_per_block, _D), lambda i: (i, 0)),
    )(x)


def _sc_body(x_hbm, out_hbm, xbuf, ukeys, obuf, hist, pbuf):
    nw = 32
    rows = x_hbm.shape[0]
    rpw = rows // nw
    wid = jax.lax.axis_index("s") * 2 + jax.lax.axis_index("c")
    base = wid * rpw
    lane = jax.lax.iota(jnp.int32, 16)
    ones16 = jnp.ones((16,), jnp.int32)
    zeros16 = jnp.zeros((16,), jnp.int32)
    true16 = jnp.full((16,), True)
    topbit = jnp.uint32(_TOPBIT)

    def zero_hist(i, _):
        hist[pl.ds(i * 16, 16)] = zeros16
        return 0

    jax.lax.fori_loop(0, 256, zero_hist, 0)

    def gquery(d):
        return jnp.sum(pbuf[pl.ds(d * 16, 16)])

    def find_digit(kp):
        pbuf[pl.ds(256 * 16, 16)] = zeros16

        def suffix(i, acc):
            d = 255 - i
            acc = acc + hist[pl.ds(d * 16, 16)]
            pbuf[pl.ds(d * 16, 16)] = acc
            hist[pl.ds(d * 16, 16)] = zeros16
            return acc

        jax.lax.fori_loop(0, 256, suffix, zeros16)

        d = jnp.int32(0)
        for bit in (128, 64, 32, 16, 8, 4, 2, 1):
            cand = d | jnp.int32(bit)
            g = gquery(cand)
            d = jnp.where(g >= kp, cand, d)
        kp_next = kp - gquery(d + 1)
        return d, kp_next

    def row_body(r, _):
        row = base + r
        pltpu.sync_copy(x_hbm.at[row], xbuf)

        def p1(vi, _):
            x = xbuf[pl.ds(vi * 16, 16)]
            bu = jax.lax.bitcast_convert_type(x, jnp.uint32)
            uk = jnp.where(bu >= topbit, ~bu, bu | topbit)
            ukeys[pl.ds(vi * 16, 16)] = uk
            digit = jax.lax.convert_element_type(uk >> jnp.uint32(24), jnp.int32)
            idx = (digit << 4) | lane
            plsc.addupdate_scatter(hist, [idx], ones16, mask=true16)
            return 0

        jax.lax.fori_loop(0, _NV, p1, 0)
        d, kp = find_digit(jnp.int32(_K))
        lo = jax.lax.convert_element_type(d, jnp.uint32) << jnp.uint32(24)

        for shift in (16, 8, 0):
            width = jnp.uint32(256 << shift)
            sh = jnp.uint32(shift)
            lo_now = lo

            def band(vi, _, lo_now=lo_now, width=width, sh=sh):
                uk = ukeys[pl.ds(vi * 16, 16)]
                rband = uk - lo_now
                m = rband < width
                digit = jax.lax.convert_element_type(rband >> sh, jnp.int32)
                idx = jnp.where(m, (digit << 4) | lane, 0)
                plsc.addupdate_scatter(hist, [idx], ones16, mask=m)
                return 0

            jax.lax.fori_loop(0, _NV, band, 0)
            d, kp = find_digit(kp)
            lo = lo | (jax.lax.convert_element_type(d, jnp.uint32) << sh)

        thr = lo

        def fin(vi, _):
            uk = ukeys[pl.ds(vi * 16, 16)]
            sel = jnp.where(uk >= thr, uk, topbit)
            bits = jnp.where(sel < topbit, ~sel, sel ^ topbit)
            obuf[pl.ds(vi * 16, 16)] = jax.lax.bitcast_convert_type(
                bits, jnp.float32)
            return 0

        jax.lax.fori_loop(0, _NV, fin, 0)
        pltpu.sync_copy(obuf, out_hbm.at[row])
        return 0

    jax.lax.fori_loop(0, rpw, row_body, 0)


def _sc_topk(x):
    mesh = plsc.VectorSubcoreMesh(core_axis_name="c", subcore_axis_name="s")
    f = functools.partial(
        pl.kernel,
        out_type=jax.ShapeDtypeStruct(x.shape, x.dtype),
        mesh=mesh,
        compiler_params=pltpu.CompilerParams(needs_layout_passes=False),
        scratch_types=[
            pltpu.VMEM((_D,), jnp.float32),
            pltpu.VMEM((_D,), jnp.uint32),
            pltpu.VMEM((_D,), jnp.float32),
            pltpu.VMEM((256 * 16,), jnp.int32),
            pltpu.VMEM((257 * 16,), jnp.int32),
        ],
    )(_sc_body)
    return f(x)


@jax.jit
def kernel(features):
    B, L, D = features.shape
    x = features.reshape(B * L, D)
    rows = x.shape[0]
    tc_rows = min(_TC_ROWS, rows)
    out_tc = _tc_topk(x[:tc_rows])
    if tc_rows < rows:
        out_sc = _sc_topk(x[tc_rows:])
        out = jnp.concatenate([out_tc, out_sc], axis=0)
    else:
        out = out_tc
    return out.reshape(B, L, D)


# speedup vs baseline: 4.3042x; 4.3042x over previous
import functools

import jax
import jax.numpy as jnp
from jax.experimental import pallas as pl
from jax.experimental.pallas import tpu as pltpu
from jax.experimental.pallas import tpu_sc as plsc

_K = 256
_D = 8192
_INT_MIN = -(2**31)
_TOPBIT = 0x80000000

_TC_ROWS = 2560


def _tc_topk_kernel(x_ref, o_ref):
    x = x_ref[...]
    b = jax.lax.bitcast_convert_type(x, jnp.int32)
    keys = jnp.where(b < 0, b ^ jnp.int32(0x7FFFFFFF), b)
    rows = x.shape[0]

    def body(j, u):
        bit = jnp.left_shift(jnp.int32(1), jnp.int32(31) - j)
        cand_u = u | bit
        cand_s = cand_u ^ jnp.int32(_INT_MIN)
        cnt = jnp.sum((keys >= cand_s).astype(jnp.int32), axis=1, keepdims=True)
        return jnp.where(cnt >= _K, cand_u, u)

    u_star = jax.lax.fori_loop(0, 32, body, jnp.zeros((rows, 1), jnp.int32))
    thr = u_star ^ jnp.int32(_INT_MIN)
    o_ref[...] = jnp.where(keys >= thr, x, jnp.float32(0.0))


def _tc_topk(x, n_rows, out_rows):
    rows_per_block = 256
    while n_rows % rows_per_block:
        rows_per_block //= 2
    grid = (n_rows // rows_per_block,)
    return pl.pallas_call(
        _tc_topk_kernel,
        out_shape=jax.ShapeDtypeStruct((out_rows, _D), x.dtype),
        grid=grid,
        in_specs=[pl.BlockSpec((rows_per_block, _D), lambda i: (i, 0))],
        out_specs=pl.BlockSpec((rows_per_block, _D), lambda i: (i, 0)),
    )(x)


def _sc_body(x_hbm, out_hbm, xbuf0, xbuf1, ukeys, obuf0, obuf1, hist, pbuf,
             isem0, isem1, osem0, osem1):
    nw = 32
    rows = out_hbm.shape[0]
    start = x_hbm.shape[0] - rows
    rpw = rows // nw
    wid = jax.lax.axis_index("s") * 2 + jax.lax.axis_index("c")
    base = wid * rpw
    lane = jax.lax.iota(jnp.int32, 16)
    ones16 = jnp.ones((16,), jnp.int32)
    zeros16 = jnp.zeros((16,), jnp.int32)
    true16 = jnp.full((16,), True)
    topbit = jnp.uint32(_TOPBIT)

    @plsc.parallel_loop(0, 256 * 16, 16, unroll=8)
    def zero_hist(i):
        hist[pl.ds(i, 16)] = zeros16

    def gquery(d):
        return jnp.sum(pbuf[pl.ds(d * 16, 16)])

    def find_digit(kp):
        pbuf[pl.ds(256 * 16, 16)] = zeros16

        def suffix(i, acc):
            d = 255 - i
            acc = acc + hist[pl.ds(d * 16, 16)]
            pbuf[pl.ds(d * 16, 16)] = acc
            hist[pl.ds(d * 16, 16)] = zeros16
            return acc

        jax.lax.fori_loop(0, 256, suffix, zeros16, unroll=8)

        d = jnp.int32(0)
        for bit in (128, 64, 32, 16, 8, 4, 2, 1):
            cand = d | jnp.int32(bit)
            g = gquery(cand)
            d = jnp.where(g >= kp, cand, d)
        kp_next = kp - gquery(d + 1)
        return d, kp_next

    def do_row(row, xb, ob, isem, osem):
        pltpu.make_async_copy(x_hbm.at[start + row], xb, isem).wait()

        @plsc.parallel_loop(0, _D, 16, unroll=16)
        def p1(i):
            x = xb[pl.ds(i, 16)]
            b = jax.lax.bitcast_convert_type(x, jnp.int32)
            xm = (b >> 31) | jnp.int32(_INT_MIN)
            uk = jax.lax.bitcast_convert_type(b ^ xm, jnp.uint32)
            ukeys[pl.ds(i, 16)] = uk
            dig4 = jax.lax.convert_element_type(uk >> jnp.uint32(20), jnp.int32)
            idx = (dig4 & jnp.int32(0xFF0)) | lane
            plsc.addupdate_scatter(hist, [idx], ones16, mask=true16)

        nxt = row + 2

        @pl.when(nxt < base + rpw)
        def _():
            pltpu.async_copy(x_hbm.at[start + nxt], xb, isem)

        d, kp = find_digit(jnp.int32(_K))
        lo = jax.lax.convert_element_type(d, jnp.uint32) << jnp.uint32(24)

        for shift in (16, 8, 0):
            width = jnp.uint32(256 << shift)
            lo_now = lo

            @plsc.parallel_loop(0, _D, 16, unroll=16)
            def band(i, lo_now=lo_now, width=width, shift=shift):
                uk = ukeys[pl.ds(i, 16)]
                rband = uk - lo_now
                m = rband < width
                if shift >= 4:
                    dig4 = rband >> jnp.uint32(shift - 4)
                else:
                    dig4 = rband << jnp.uint32(4 - shift)
                idx = (jax.lax.convert_element_type(dig4, jnp.int32)
                       & jnp.int32(0xFF0)) | lane
                plsc.addupdate_scatter(hist, [idx], ones16, mask=m)
            d, kp = find_digit(kp)
            lo = lo | (jax.lax.convert_element_type(d, jnp.uint32)
                       << jnp.uint32(shift))

        thr = lo

        @pl.when(row - 2 >= base)
        def _():
            pltpu.make_async_copy(ob, out_hbm.at[row], osem).wait()

        @plsc.parallel_loop(0, _D, 16, unroll=16)
        def fin(i):
            uk = ukeys[pl.ds(i, 16)]
            sel = jnp.where(uk >= thr, uk, topbit)
            bits = jnp.where(sel < topbit, ~sel, sel ^ topbit)
            ob[pl.ds(i, 16)] = jax.lax.bitcast_convert_type(
                bits, jnp.float32)
        pltpu.async_copy(ob, out_hbm.at[row], osem)

    bufs = (
        (xbuf0, obuf0, isem0, osem0),
        (xbuf1, obuf1, isem1, osem1),
    )
    pltpu.async_copy(x_hbm.at[start + base], bufs[0][0], bufs[0][2])
    pltpu.async_copy(x_hbm.at[start + base + 1], bufs[1][0], bufs[1][2])

    def row_pair(p, _):
        r0 = base + 2 * p
        for b in (0, 1):
            do_row(r0 + b, *bufs[b])
        return 0

    jax.lax.fori_loop(0, rpw // 2, row_pair, 0)
    for b in (0, 1):
        last = base + rpw - 2 + b
        pltpu.make_async_copy(bufs[b][1], out_hbm.at[last], bufs[b][3]).wait()


def _sc_topk(x, n_rows):
    mesh = plsc.VectorSubcoreMesh(core_axis_name="c", subcore_axis_name="s")
    f = functools.partial(
        pl.kernel,
        out_type=jax.ShapeDtypeStruct((n_rows, _D), x.dtype),
        mesh=mesh,
        compiler_params=pltpu.CompilerParams(needs_layout_passes=False),
        scratch_types=[
            pltpu.VMEM((_D,), jnp.float32),
            pltpu.VMEM((_D,), jnp.float32),
            pltpu.VMEM((_D,), jnp.uint32),
            pltpu.VMEM((_D,), jnp.float32),
            pltpu.VMEM((_D,), jnp.float32),
            pltpu.VMEM((256 * 16,), jnp.int32),
            pltpu.VMEM((257 * 16,), jnp.int32),
            pltpu.SemaphoreType.DMA,
            pltpu.SemaphoreType.DMA,
            pltpu.SemaphoreType.DMA,
            pltpu.SemaphoreType.DMA,
        ],
    )(_sc_body)
    return f(x)


@jax.jit
def kernel(features):
    B, L, D = features.shape
    x = features.reshape(B * L, D)
    rows = x.shape[0]
    tc_rows = min(_TC_ROWS, rows)
    out_tc = _tc_topk(x, tc_rows, rows)
    if tc_rows < rows:
        out_sc = _sc_topk(x, rows - tc_rows)
        out = jax.lax.dynamic_update_slice(out_tc, out_sc, (tc_rows, 0))
    else:
        out = out_tc
    return out.reshape(B, L, D)
